# baseline (speedup 1.0000x reference)
import functools

import jax
import jax.numpy as jnp
from jax import lax
from jax.experimental import pallas as pl
from jax.experimental.pallas import tpu as pltpu
from jax.experimental.pallas import tpu_sc as plsc

N = 10000
E = 160000
D = 256
DH = 128
NG = 16
NCORE = 2
NSUB = 16
CH = 128
NCHUNK_P = 1280
NJT = NCHUNK_P // NSUB
BK = 40
NB = NJT // BK
NP = 2 * N + 16
R_SMALL = 624
R_LAST = N - (NSUB - 1) * R_SMALL

@functools.lru_cache(maxsize=None)
def _mesh():
    return plsc.VectorSubcoreMesh(
        core_axis_name="c", subcore_axis_name="s",
        num_cores=NCORE, num_subcores=NSUB)


def _sc_agg_body(u_hbm, src_hbm, dst_hbm, y_hbm,
                 srcv, dstv, rows, srcv2, dstv2, rows2, acc_sh, sems):
    c = lax.axis_index("c")
    s = lax.axis_index("s")
    r0 = s * R_SMALL

    def _init(nrows):
        pltpu.sync_copy(u_hbm.at[pl.ds(c * N + r0, nrows)],
                        acc_sh.at[pl.ds(r0, nrows)])

    @pl.when(s < NSUB - 1)
    def _():
        _init(R_SMALL)

    @pl.when(s == NSUB - 1)
    def _():
        _init(R_LAST)

    plsc.subcore_barrier()
    e0 = s * NJT * CH
    base = c * NCHUNK_P * CH + e0

    def pair_step(j, carry):
        ea = pl.multiple_of(2 * j * CH, CH)
        pltpu.sync_copy(src_hbm.at[pl.ds(base + ea, CH)], srcv)
        pltpu.sync_copy(dst_hbm.at[pl.ds(e0 + ea, CH)], dstv)
        ga = pltpu.async_copy(u_hbm.at[srcv], rows, sems.at[0])
        pltpu.sync_copy(src_hbm.at[pl.ds(base + ea + CH, CH)], srcv2)
        pltpu.sync_copy(dst_hbm.at[pl.ds(e0 + ea + CH, CH)], dstv2)
        gb = pltpu.async_copy(u_hbm.at[srcv2], rows2, sems.at[1])
        ga.wait()
        pltpu.sync_copy(rows, acc_sh.at[dstv], add=True)
        gb.wait()
        pltpu.sync_copy(rows2, acc_sh.at[dstv2], add=True)
        return carry

    lax.fori_loop(0, NJT // 2, pair_step, 0)
    plsc.subcore_barrier()

    def _wb(nrows):
        pltpu.sync_copy(acc_sh.at[pl.ds(r0, nrows)],
                        y_hbm.at[pl.ds(c * N + r0, nrows)])

    @pl.when(s < NSUB - 1)
    def _():
        _wb(R_SMALL)

    @pl.when(s == NSUB - 1)
    def _():
        _wb(R_LAST)


@functools.lru_cache(maxsize=None)
def _sc_agg():
    return pl.kernel(
        _sc_agg_body,
        out_type=jax.ShapeDtypeStruct((NCORE * N, DH), jnp.float32),
        mesh=_mesh(),
        scratch_types=[
            pltpu.VMEM((CH,), jnp.int32),
            pltpu.VMEM((CH,), jnp.int32),
            pltpu.VMEM((CH, DH), jnp.float32),
            pltpu.VMEM((CH,), jnp.int32),
            pltpu.VMEM((CH,), jnp.int32),
            pltpu.VMEM((CH, DH), jnp.float32),
            pltpu.VMEM_SHARED((N, DH), jnp.float32),
            pltpu.SemaphoreType.DMA((2,)),
        ],
    )


def _split_halves(u_ref, u):
    u_ref[0:N, :] = u[:, 0:DH]
    u_ref[N:2 * N, :] = u[:, DH:D]
    u_ref[2 * N:NP, :] = jnp.zeros((NP - 2 * N, DH), jnp.float32)


def _tc_enc_body(x_ref, w_ref, degp_ref, u_ref, dis_ref):
    deg = degp_ref[0:N, 0:1]
    dis = lax.rsqrt(deg)
    dis_ref[...] = dis
    h = jnp.dot(x_ref[...], w_ref[...], preferred_element_type=jnp.float32)
    _split_halves(u_ref, h * dis)


_tc_enc = pl.pallas_call(
    _tc_enc_body,
    out_shape=[jax.ShapeDtypeStruct((NP, DH), jnp.float32),
               jax.ShapeDtypeStruct((N, 1), jnp.float32)],
)


def _tc_mid_body(has_bn, y_ref, dis_ref, b_ref, w_ref, g_ref, be_ref, u_ref):
    dis = dis_ref[...]
    h = jnp.concatenate([y_ref[0:N, :], y_ref[N:2 * N, :]], axis=1)
    h = h * dis + b_ref[...]
    if has_bn:
        m = jnp.mean(h, axis=0, keepdims=True)
        v = jnp.mean((h - m) ** 2, axis=0, keepdims=True)
        h = (h - m) * lax.rsqrt(v + 1e-5) * g_ref[...] + be_ref[...]
        h = jnp.maximum(h, 0.0)
    h2 = jnp.dot(h, w_ref[...], preferred_element_type=jnp.float32)
    _split_halves(u_ref, h2 * dis)


_tc_mid_bn = pl.pallas_call(
    functools.partial(_tc_mid_body, True),
    out_shape=jax.ShapeDtypeStruct((NP, DH), jnp.float32),
)
_tc_mid_plain = pl.pallas_call(
    functools.partial(_tc_mid_body, False),
    out_shape=jax.ShapeDtypeStruct((NP, DH), jnp.float32),
)


def _tc_fin_body(y_ref, dis_ref, b_ref, batch_ref, wc_ref, bc_ref, out_ref):
    dis = dis_ref[...]
    h = jnp.concatenate([y_ref[0:N, :], y_ref[N:2 * N, :]], axis=1)
    h = h * dis + b_ref[...]
    onehot = (batch_ref[...] == lax.broadcasted_iota(jnp.int32, (N, NG), 1)
              ).astype(jnp.float32)
    ssum = lax.dot_general(onehot, h, (((0,), (0,)), ((), ())),
                           preferred_element_type=jnp.float32)
    cnt = jnp.sum(onehot, axis=0, keepdims=True)
    pooled = ssum / jnp.maximum(cnt.T, 1.0)
    out_ref[...] = jnp.dot(pooled, wc_ref[...],
                           preferred_element_type=jnp.float32) + bc_ref[...]


_tc_fin = pl.pallas_call(
    _tc_fin_body,
    out_shape=jax.ShapeDtypeStruct((NG, 64), jnp.float32),
)


def kernel(x, edge_index, batch, edge_attr, W_enc, b_enc, W_convs, b_convs,
           gamma, beta, W_clf, b_clf):
    src = edge_index[0].astype(jnp.int32)
    dst = edge_index[1].astype(jnp.int32)
    npad = NCHUNK_P * CH - E
    pad_src = jnp.full((npad,), 2 * N, jnp.int32)
    pad_dst = (jnp.arange(npad, dtype=jnp.int32) * 13) % N
    src2 = jnp.concatenate([src, pad_src, src + N, pad_src])
    dst2 = jnp.concatenate([dst, pad_dst])

    sc_agg = _sc_agg()
    ones_u = jnp.concatenate([jnp.ones((2 * N, DH), jnp.float32),
                              jnp.zeros((NP - 2 * N, DH), jnp.float32)])
    degp = sc_agg(ones_u, src2, dst2)
    u, dis = _tc_enc(x, W_enc, degp)

    y = sc_agg(u, src2, dst2)
    u = _tc_mid_plain(y, dis, b_enc.reshape(1, D), W_convs[0],
                      gamma[0].reshape(1, D), beta[0].reshape(1, D))
    y = sc_agg(u, src2, dst2)
    u = _tc_mid_bn(y, dis, b_convs[0].reshape(1, D), W_convs[2],
                   gamma[0].reshape(1, D), beta[0].reshape(1, D))
    y = sc_agg(u, src2, dst2)
    u = _tc_mid_plain(y, dis, b_convs[2].reshape(1, D), W_convs[1],
                      gamma[0].reshape(1, D), beta[0].reshape(1, D))
    y = sc_agg(u, src2, dst2)
    u = _tc_mid_bn(y, dis, b_convs[1].reshape(1, D), W_convs[2],
                   gamma[1].reshape(1, D), beta[1].reshape(1, D))
    y = sc_agg(u, src2, dst2)
    out = _tc_fin(y, dis, b_convs[2].reshape(1, D),
                  batch.astype(jnp.int32).reshape(N, 1), W_clf,
                  b_clf.reshape(1, 64))
    return out


# speedup vs baseline: 1.7084x; 1.7084x over previous
import functools

import jax
import jax.numpy as jnp
from jax import lax
from jax.experimental import pallas as pl
from jax.experimental.pallas import tpu as pltpu
from jax.experimental.pallas import tpu_sc as plsc

N = 10000
E = 160000
D = 256
DH = 128
NG = 16
NCORE = 2
NSUB = 16
CH = 128
NCHUNK_P = 1280
NJT = NCHUNK_P // NSUB
BK = 40
NB = NJT // BK
NP = 2 * N + 16
R_SMALL = 624
R_LAST = N - (NSUB - 1) * R_SMALL

@functools.lru_cache(maxsize=None)
def _mesh():
    return plsc.VectorSubcoreMesh(
        core_axis_name="c", subcore_axis_name="s",
        num_cores=NCORE, num_subcores=NSUB)


def _sc_agg_body(u_hbm, src_hbm, dst_hbm, y_hbm,
                 srcv, dstv, rows, acc_sh, sems):
    c = lax.axis_index("c")
    s = lax.axis_index("s")
    r0 = s * R_SMALL

    def _init(nrows):
        pltpu.sync_copy(u_hbm.at[pl.ds(c * N + r0, nrows)],
                        acc_sh.at[pl.ds(r0, nrows)])

    @pl.when(s < NSUB - 1)
    def _():
        _init(R_SMALL)

    @pl.when(s == NSUB - 1)
    def _():
        _init(R_LAST)

    plsc.subcore_barrier()
    nchunk = E // CH

    def chunk_step(j, carry):
        chunk = j * NSUB + s

        @pl.when(chunk < nchunk)
        def _():
            off = pl.multiple_of(chunk * CH, CH)
            pltpu.sync_copy(src_hbm.at[pl.ds(c * NCHUNK_P * CH + off, CH)], srcv)
            pltpu.sync_copy(dst_hbm.at[pl.ds(off, CH)], dstv)
            pltpu.async_copy(u_hbm.at[srcv], rows, sems).wait()
            pltpu.sync_copy(rows, acc_sh.at[dstv], add=True)
        return carry

    lax.fori_loop(0, (nchunk + NSUB - 1) // NSUB, chunk_step, 0)
    plsc.subcore_barrier()

    def _wb(nrows):
        pltpu.sync_copy(acc_sh.at[pl.ds(r0, nrows)],
                        y_hbm.at[pl.ds(c * N + r0, nrows)])

    @pl.when(s < NSUB - 1)
    def _():
        _wb(R_SMALL)

    @pl.when(s == NSUB - 1)
    def _():
        _wb(R_LAST)


@functools.lru_cache(maxsize=None)
def _sc_agg():
    return pl.kernel(
        _sc_agg_body,
        out_type=jax.ShapeDtypeStruct((NCORE * N, DH), jnp.float32),
        mesh=_mesh(),
        scratch_types=[
            pltpu.VMEM((CH,), jnp.int32),
            pltpu.VMEM((CH,), jnp.int32),
            pltpu.VMEM((CH, DH), jnp.float32),
            pltpu.VMEM_SHARED((N, DH), jnp.float32),
            pltpu.SemaphoreType.DMA,
        ],
    )


def _split_halves(u_ref, u):
    u_ref[0:N, :] = u[:, 0:DH]
    u_ref[N:2 * N, :] = u[:, DH:D]
    u_ref[2 * N:NP, :] = jnp.zeros((NP - 2 * N, DH), jnp.float32)


def _tc_enc_body(x_ref, w_ref, degp_ref, u_ref, dis_ref):
    deg = degp_ref[0:N, 0:1]
    dis = lax.rsqrt(deg)
    dis_ref[...] = dis
    h = jnp.dot(x_ref[...], w_ref[...], preferred_element_type=jnp.float32)
    _split_halves(u_ref, h * dis)


_tc_enc = pl.pallas_call(
    _tc_enc_body,
    out_shape=[jax.ShapeDtypeStruct((NP, DH), jnp.float32),
               jax.ShapeDtypeStruct((N, 1), jnp.float32)],
)


def _tc_mid_body(has_bn, y_ref, dis_ref, b_ref, w_ref, g_ref, be_ref, u_ref):
    dis = dis_ref[...]
    h = jnp.concatenate([y_ref[0:N, :], y_ref[N:2 * N, :]], axis=1)
    h = h * dis + b_ref[...]
    if has_bn:
        m = jnp.mean(h, axis=0, keepdims=True)
        v = jnp.mean((h - m) ** 2, axis=0, keepdims=True)
        h = (h - m) * lax.rsqrt(v + 1e-5) * g_ref[...] + be_ref[...]
        h = jnp.maximum(h, 0.0)
    h2 = jnp.dot(h, w_ref[...], preferred_element_type=jnp.float32)
    _split_halves(u_ref, h2 * dis)


_tc_mid_bn = pl.pallas_call(
    functools.partial(_tc_mid_body, True),
    out_shape=jax.ShapeDtypeStruct((NP, DH), jnp.float32),
)
_tc_mid_plain = pl.pallas_call(
    functools.partial(_tc_mid_body, False),
    out_shape=jax.ShapeDtypeStruct((NP, DH), jnp.float32),
)


def _tc_fin_body(y_ref, dis_ref, b_ref, batch_ref, wc_ref, bc_ref, out_ref):
    dis = dis_ref[...]
    h = jnp.concatenate([y_ref[0:N, :], y_ref[N:2 * N, :]], axis=1)
    h = h * dis + b_ref[...]
    onehot = (batch_ref[...] == lax.broadcasted_iota(jnp.int32, (N, NG), 1)
              ).astype(jnp.float32)
    ssum = lax.dot_general(onehot, h, (((0,), (0,)), ((), ())),
                           preferred_element_type=jnp.float32)
    cnt = jnp.sum(onehot, axis=0, keepdims=True)
    pooled = ssum / jnp.maximum(cnt.T, 1.0)
    out_ref[...] = jnp.dot(pooled, wc_ref[...],
                           preferred_element_type=jnp.float32) + bc_ref[...]


_tc_fin = pl.pallas_call(
    _tc_fin_body,
    out_shape=jax.ShapeDtypeStruct((NG, 64), jnp.float32),
)


def kernel(x, edge_index, batch, edge_attr, W_enc, b_enc, W_convs, b_convs,
           gamma, beta, W_clf, b_clf):
    src = edge_index[0].astype(jnp.int32)
    dst = edge_index[1].astype(jnp.int32)
    npad = NCHUNK_P * CH - E
    pad_src = jnp.full((npad,), 2 * N, jnp.int32)
    pad_dst = (jnp.arange(npad, dtype=jnp.int32) * 13) % N
    src2 = jnp.concatenate([src, pad_src, src + N, pad_src])
    dst2 = jnp.concatenate([dst, pad_dst])

    sc_agg = _sc_agg()
    ones_u = jnp.concatenate([jnp.ones((2 * N, DH), jnp.float32),
                              jnp.zeros((NP - 2 * N, DH), jnp.float32)])
    degp = sc_agg(ones_u, src2, dst2)
    u, dis = _tc_enc(x, W_enc, degp)

    y = sc_agg(u, src2, dst2)
    u = _tc_mid_plain(y, dis, b_enc.reshape(1, D), W_convs[0],
                      gamma[0].reshape(1, D), beta[0].reshape(1, D))
    y = sc_agg(u, src2, dst2)
    u = _tc_mid_bn(y, dis, b_convs[0].reshape(1, D), W_convs[2],
                   gamma[0].reshape(1, D), beta[0].reshape(1, D))
    y = sc_agg(u, src2, dst2)
    u = _tc_mid_plain(y, dis, b_convs[2].reshape(1, D), W_convs[1],
                      gamma[0].reshape(1, D), beta[0].reshape(1, D))
    y = sc_agg(u, src2, dst2)
    u = _tc_mid_bn(y, dis, b_convs[1].reshape(1, D), W_convs[2],
                   gamma[1].reshape(1, D), beta[1].reshape(1, D))
    y = sc_agg(u, src2, dst2)
    out = _tc_fin(y, dis, b_convs[2].reshape(1, D),
                  batch.astype(jnp.int32).reshape(N, 1), W_clf,
                  b_clf.reshape(1, 64))
    return out


# speedup vs baseline: 2.2139x; 1.2959x over previous
import functools

import jax
import jax.numpy as jnp
from jax import lax
from jax.experimental import pallas as pl
from jax.experimental.pallas import tpu as pltpu
from jax.experimental.pallas import tpu_sc as plsc

N = 10000
E = 160000
D = 256
DH = 128
NG = 16
NCORE = 2
NSUB = 16
CH = 256
EPC = 163840
NP = 2 * N + 16
R_SMALL = 624
R_LAST = N - (NSUB - 1) * R_SMALL

@functools.lru_cache(maxsize=None)
def _mesh():
    return plsc.VectorSubcoreMesh(
        core_axis_name="c", subcore_axis_name="s",
        num_cores=NCORE, num_subcores=NSUB)


def _sc_agg_body(u_hbm, src_hbm, dst_hbm, y_hbm,
                 srcv, dstv, rows, acc_sh, sems):
    c = lax.axis_index("c")
    s = lax.axis_index("s")
    r0 = s * R_SMALL

    def _init(nrows):
        pltpu.sync_copy(u_hbm.at[pl.ds(c * N + r0, nrows)],
                        acc_sh.at[pl.ds(r0, nrows)])

    @pl.when(s < NSUB - 1)
    def _():
        _init(R_SMALL)

    @pl.when(s == NSUB - 1)
    def _():
        _init(R_LAST)

    plsc.subcore_barrier()
    nchunk = E // CH

    def chunk_step(j, carry):
        chunk = j * NSUB + s

        @pl.when(chunk < nchunk)
        def _():
            off = pl.multiple_of(chunk * CH, CH)
            pltpu.sync_copy(src_hbm.at[pl.ds(c * EPC + off, CH)], srcv)
            pltpu.sync_copy(dst_hbm.at[pl.ds(off, CH)], dstv)
            pltpu.async_copy(u_hbm.at[srcv], rows, sems).wait()
            pltpu.sync_copy(rows, acc_sh.at[dstv], add=True)
        return carry

    lax.fori_loop(0, (nchunk + NSUB - 1) // NSUB, chunk_step, 0)
    plsc.subcore_barrier()

    def _wb(nrows):
        pltpu.sync_copy(acc_sh.at[pl.ds(r0, nrows)],
                        y_hbm.at[pl.ds(c * N + r0, nrows)])

    @pl.when(s < NSUB - 1)
    def _():
        _wb(R_SMALL)

    @pl.when(s == NSUB - 1)
    def _():
        _wb(R_LAST)


@functools.lru_cache(maxsize=None)
def _sc_agg():
    return pl.kernel(
        _sc_agg_body,
        out_type=jax.ShapeDtypeStruct((NCORE * N, DH), jnp.float32),
        mesh=_mesh(),
        scratch_types=[
            pltpu.VMEM((CH,), jnp.int32),
            pltpu.VMEM((CH,), jnp.int32),
            pltpu.VMEM((CH, DH), jnp.float32),
            pltpu.VMEM_SHARED((N, DH), jnp.float32),
            pltpu.SemaphoreType.DMA,
        ],
    )


def _split_halves(u_ref, u):
    u_ref[0:N, :] = u[:, 0:DH]
    u_ref[N:2 * N, :] = u[:, DH:D]
    u_ref[2 * N:NP, :] = jnp.zeros((NP - 2 * N, DH), jnp.float32)


def _tc_enc_body(x_ref, w_ref, degp_ref, u_ref, dis_ref):
    deg = degp_ref[0:N, 0:1]
    dis = lax.rsqrt(deg)
    dis_ref[...] = dis
    h = jnp.dot(x_ref[...], w_ref[...], preferred_element_type=jnp.float32)
    _split_halves(u_ref, h * dis)


_tc_enc = pl.pallas_call(
    _tc_enc_body,
    out_shape=[jax.ShapeDtypeStruct((NP, DH), jnp.float32),
               jax.ShapeDtypeStruct((N, 1), jnp.float32)],
)


def _tc_mid_body(has_bn, y_ref, dis_ref, b_ref, w_ref, g_ref, be_ref, u_ref):
    dis = dis_ref[...]
    h = jnp.concatenate([y_ref[0:N, :], y_ref[N:2 * N, :]], axis=1)
    h = h * dis + b_ref[...]
    if has_bn:
        m = jnp.mean(h, axis=0, keepdims=True)
        v = jnp.mean((h - m) ** 2, axis=0, keepdims=True)
        h = (h - m) * lax.rsqrt(v + 1e-5) * g_ref[...] + be_ref[...]
        h = jnp.maximum(h, 0.0)
    h2 = jnp.dot(h, w_ref[...], preferred_element_type=jnp.float32)
    _split_halves(u_ref, h2 * dis)


_tc_mid_bn = pl.pallas_call(
    functools.partial(_tc_mid_body, True),
    out_shape=jax.ShapeDtypeStruct((NP, DH), jnp.float32),
)
_tc_mid_plain = pl.pallas_call(
    functools.partial(_tc_mid_body, False),
    out_shape=jax.ShapeDtypeStruct((NP, DH), jnp.float32),
)


def _tc_fin_body(y_ref, dis_ref, b_ref, batch_ref, wc_ref, bc_ref, out_ref):
    dis = dis_ref[...]
    h = jnp.concatenate([y_ref[0:N, :], y_ref[N:2 * N, :]], axis=1)
    h = h * dis + b_ref[...]
    onehot = (batch_ref[...] == lax.broadcasted_iota(jnp.int32, (N, NG), 1)
              ).astype(jnp.float32)
    ssum = lax.dot_general(onehot, h, (((0,), (0,)), ((), ())),
                           preferred_element_type=jnp.float32)
    cnt = jnp.sum(onehot, axis=0, keepdims=True)
    pooled = ssum / jnp.maximum(cnt.T, 1.0)
    out_ref[...] = jnp.dot(pooled, wc_ref[...],
                           preferred_element_type=jnp.float32) + bc_ref[...]


_tc_fin = pl.pallas_call(
    _tc_fin_body,
    out_shape=jax.ShapeDtypeStruct((NG, 64), jnp.float32),
)


def kernel(x, edge_index, batch, edge_attr, W_enc, b_enc, W_convs, b_convs,
           gamma, beta, W_clf, b_clf):
    src = edge_index[0].astype(jnp.int32)
    dst = edge_index[1].astype(jnp.int32)
    npad = EPC - E
    pad_src = jnp.full((npad,), 2 * N, jnp.int32)
    pad_dst = (jnp.arange(npad, dtype=jnp.int32) * 13) % N
    src2 = jnp.concatenate([src, pad_src, src + N, pad_src])
    dst2 = jnp.concatenate([dst, pad_dst])

    sc_agg = _sc_agg()
    ones_u = jnp.concatenate([jnp.ones((2 * N, DH), jnp.float32),
                              jnp.zeros((NP - 2 * N, DH), jnp.float32)])
    degp = sc_agg(ones_u, src2, dst2)
    u, dis = _tc_enc(x, W_enc, degp)

    y = sc_agg(u, src2, dst2)
    u = _tc_mid_plain(y, dis, b_enc.reshape(1, D), W_convs[0],
                      gamma[0].reshape(1, D), beta[0].reshape(1, D))
    y = sc_agg(u, src2, dst2)
    u = _tc_mid_bn(y, dis, b_convs[0].reshape(1, D), W_convs[2],
                   gamma[0].reshape(1, D), beta[0].reshape(1, D))
    y = sc_agg(u, src2, dst2)
    u = _tc_mid_plain(y, dis, b_convs[2].reshape(1, D), W_convs[1],
                      gamma[0].reshape(1, D), beta[0].reshape(1, D))
    y = sc_agg(u, src2, dst2)
    u = _tc_mid_bn(y, dis, b_convs[1].reshape(1, D), W_convs[2],
                   gamma[1].reshape(1, D), beta[1].reshape(1, D))
    y = sc_agg(u, src2, dst2)
    out = _tc_fin(y, dis, b_convs[2].reshape(1, D),
                  batch.astype(jnp.int32).reshape(N, 1), W_clf,
                  b_clf.reshape(1, 64))
    return out


# speedup vs baseline: 2.2581x; 1.0200x over previous
import functools

import jax
import jax.numpy as jnp
from jax import lax
from jax.experimental import pallas as pl
from jax.experimental.pallas import tpu as pltpu
from jax.experimental.pallas import tpu_sc as plsc

N = 10000
E = 160000
D = 256
DH = 128
NG = 16
NCORE = 2
NSUB = 16
CH = 384
EPC = 163840
NP = 2 * N + 16
R_SMALL = 624
R_LAST = N - (NSUB - 1) * R_SMALL

@functools.lru_cache(maxsize=None)
def _mesh():
    return plsc.VectorSubcoreMesh(
        core_axis_name="c", subcore_axis_name="s",
        num_cores=NCORE, num_subcores=NSUB)


def _sc_agg_body(u_hbm, src_hbm, dst_hbm, y_hbm,
                 srcv, dstv, rows, acc_sh, sems):
    c = lax.axis_index("c")
    s = lax.axis_index("s")
    r0 = s * R_SMALL

    def _init(nrows):
        pltpu.sync_copy(u_hbm.at[pl.ds(c * N + r0, nrows)],
                        acc_sh.at[pl.ds(r0, nrows)])

    @pl.when(s < NSUB - 1)
    def _():
        _init(R_SMALL)

    @pl.when(s == NSUB - 1)
    def _():
        _init(R_LAST)

    plsc.subcore_barrier()
    nchunk = (E + CH - 1) // CH

    def chunk_step(j, carry):
        chunk = j * NSUB + s

        @pl.when(chunk < nchunk)
        def _():
            off = pl.multiple_of(chunk * CH, CH)
            pltpu.sync_copy(src_hbm.at[pl.ds(c * EPC + off, CH)], srcv)
            pltpu.sync_copy(dst_hbm.at[pl.ds(off, CH)], dstv)
            pltpu.async_copy(u_hbm.at[srcv], rows, sems).wait()
            pltpu.sync_copy(rows, acc_sh.at[dstv], add=True)
        return carry

    lax.fori_loop(0, (nchunk + NSUB - 1) // NSUB, chunk_step, 0)
    plsc.subcore_barrier()

    def _wb(nrows):
        pltpu.sync_copy(acc_sh.at[pl.ds(r0, nrows)],
                        y_hbm.at[pl.ds(c * N + r0, nrows)])

    @pl.when(s < NSUB - 1)
    def _():
        _wb(R_SMALL)

    @pl.when(s == NSUB - 1)
    def _():
        _wb(R_LAST)


@functools.lru_cache(maxsize=None)
def _sc_agg():
    return pl.kernel(
        _sc_agg_body,
        out_type=jax.ShapeDtypeStruct((NCORE * N, DH), jnp.float32),
        mesh=_mesh(),
        scratch_types=[
            pltpu.VMEM((CH,), jnp.int32),
            pltpu.VMEM((CH,), jnp.int32),
            pltpu.VMEM((CH, DH), jnp.float32),
            pltpu.VMEM_SHARED((N, DH), jnp.float32),
            pltpu.SemaphoreType.DMA,
        ],
    )


def _split_halves(u_ref, u):
    u_ref[0:N, :] = u[:, 0:DH]
    u_ref[N:2 * N, :] = u[:, DH:D]
    u_ref[2 * N:NP, :] = jnp.zeros((NP - 2 * N, DH), jnp.float32)


def _tc_enc_body(x_ref, w_ref, degp_ref, u_ref, dis_ref):
    deg = degp_ref[0:N, 0:1]
    dis = lax.rsqrt(deg)
    dis_ref[...] = dis
    h = jnp.dot(x_ref[...], w_ref[...], preferred_element_type=jnp.float32)
    _split_halves(u_ref, h * dis)


_tc_enc = pl.pallas_call(
    _tc_enc_body,
    out_shape=[jax.ShapeDtypeStruct((NP, DH), jnp.float32),
               jax.ShapeDtypeStruct((N, 1), jnp.float32)],
)


def _tc_mid_body(has_bn, y_ref, dis_ref, b_ref, w_ref, g_ref, be_ref, u_ref):
    dis = dis_ref[...]
    h = jnp.concatenate([y_ref[0:N, :], y_ref[N:2 * N, :]], axis=1)
    h = h * dis + b_ref[...]
    if has_bn:
        m = jnp.mean(h, axis=0, keepdims=True)
        v = jnp.mean((h - m) ** 2, axis=0, keepdims=True)
        h = (h - m) * lax.rsqrt(v + 1e-5) * g_ref[...] + be_ref[...]
        h = jnp.maximum(h, 0.0)
    h2 = jnp.dot(h, w_ref[...], preferred_element_type=jnp.float32)
    _split_halves(u_ref, h2 * dis)


_tc_mid_bn = pl.pallas_call(
    functools.partial(_tc_mid_body, True),
    out_shape=jax.ShapeDtypeStruct((NP, DH), jnp.float32),
)
_tc_mid_plain = pl.pallas_call(
    functools.partial(_tc_mid_body, False),
    out_shape=jax.ShapeDtypeStruct((NP, DH), jnp.float32),
)


def _tc_fin_body(y_ref, dis_ref, b_ref, batch_ref, wc_ref, bc_ref, out_ref):
    dis = dis_ref[...]
    h = jnp.concatenate([y_ref[0:N, :], y_ref[N:2 * N, :]], axis=1)
    h = h * dis + b_ref[...]
    onehot = (batch_ref[...] == lax.broadcasted_iota(jnp.int32, (N, NG), 1)
              ).astype(jnp.float32)
    ssum = lax.dot_general(onehot, h, (((0,), (0,)), ((), ())),
                           preferred_element_type=jnp.float32)
    cnt = jnp.sum(onehot, axis=0, keepdims=True)
    pooled = ssum / jnp.maximum(cnt.T, 1.0)
    out_ref[...] = jnp.dot(pooled, wc_ref[...],
                           preferred_element_type=jnp.float32) + bc_ref[...]


_tc_fin = pl.pallas_call(
    _tc_fin_body,
    out_shape=jax.ShapeDtypeStruct((NG, 64), jnp.float32),
)


def kernel(x, edge_index, batch, edge_attr, W_enc, b_enc, W_convs, b_convs,
           gamma, beta, W_clf, b_clf):
    src = edge_index[0].astype(jnp.int32)
    dst = edge_index[1].astype(jnp.int32)
    npad = EPC - E
    pad_src = jnp.full((npad,), 2 * N, jnp.int32)
    pad_dst = (jnp.arange(npad, dtype=jnp.int32) * 13) % N
    src2 = jnp.concatenate([src, pad_src, src + N, pad_src])
    dst2 = jnp.concatenate([dst, pad_dst])

    sc_agg = _sc_agg()
    ones_u = jnp.concatenate([jnp.ones((2 * N, DH), jnp.float32),
                              jnp.zeros((NP - 2 * N, DH), jnp.float32)])
    degp = sc_agg(ones_u, src2, dst2)
    u, dis = _tc_enc(x, W_enc, degp)

    y = sc_agg(u, src2, dst2)
    u = _tc_mid_plain(y, dis, b_enc.reshape(1, D), W_convs[0],
                      gamma[0].reshape(1, D), beta[0].reshape(1, D))
    y = sc_agg(u, src2, dst2)
    u = _tc_mid_bn(y, dis, b_convs[0].reshape(1, D), W_convs[2],
                   gamma[0].reshape(1, D), beta[0].reshape(1, D))
    y = sc_agg(u, src2, dst2)
    u = _tc_mid_plain(y, dis, b_convs[2].reshape(1, D), W_convs[1],
                      gamma[0].reshape(1, D), beta[0].reshape(1, D))
    y = sc_agg(u, src2, dst2)
    u = _tc_mid_bn(y, dis, b_convs[1].reshape(1, D), W_convs[2],
                   gamma[1].reshape(1, D), beta[1].reshape(1, D))
    y = sc_agg(u, src2, dst2)
    out = _tc_fin(y, dis, b_convs[2].reshape(1, D),
                  batch.astype(jnp.int32).reshape(N, 1), W_clf,
                  b_clf.reshape(1, 64))
    return out


# speedup vs baseline: 2.5734x; 1.1396x over previous
import functools

import jax
import jax.numpy as jnp
from jax import lax
from jax.experimental import pallas as pl
from jax.experimental.pallas import tpu as pltpu
from jax.experimental.pallas import tpu_sc as plsc

N = 10000
E = 160000
D = 256
DH = 128
NG = 16
NCORE = 2
NSUB = 16
CH = 192
EPC = 163840
NP = 2 * N + 16
R_SMALL = 624
R_LAST = N - (NSUB - 1) * R_SMALL

@functools.lru_cache(maxsize=None)
def _mesh():
    return plsc.VectorSubcoreMesh(
        core_axis_name="c", subcore_axis_name="s",
        num_cores=NCORE, num_subcores=NSUB)


def _sc_agg_body(u_hbm, src_hbm, dst_hbm, y_hbm,
                 srcv, dstv, rows, srcv2, dstv2, rows2, acc_sh, sem_a, sem_b):
    c = lax.axis_index("c")
    s = lax.axis_index("s")
    r0 = s * R_SMALL

    def _init(nrows):
        pltpu.sync_copy(u_hbm.at[pl.ds(c * N + r0, nrows)],
                        acc_sh.at[pl.ds(r0, nrows)])

    @pl.when(s < NSUB - 1)
    def _():
        _init(R_SMALL)

    @pl.when(s == NSUB - 1)
    def _():
        _init(R_LAST)

    plsc.subcore_barrier()
    nchunk = (E + CH - 1) // CH

    def pair_step(j, carry):
        ca = (2 * j) * NSUB + s
        cb = (2 * j + 1) * NSUB + s

        @pl.when(ca < nchunk)
        def _():
            off = pl.multiple_of(ca * CH, CH)
            pltpu.sync_copy(src_hbm.at[pl.ds(c * EPC + off, CH)], srcv)
            pltpu.sync_copy(dst_hbm.at[pl.ds(off, CH)], dstv)
            ga = pltpu.async_copy(u_hbm.at[srcv], rows, sem_a)

            @pl.when(cb < nchunk)
            def _():
                off2 = pl.multiple_of(cb * CH, CH)
                pltpu.sync_copy(src_hbm.at[pl.ds(c * EPC + off2, CH)], srcv2)
                pltpu.sync_copy(dst_hbm.at[pl.ds(off2, CH)], dstv2)
                gb = pltpu.async_copy(u_hbm.at[srcv2], rows2, sem_b)
                ga.wait()
                pltpu.sync_copy(rows, acc_sh.at[dstv], add=True)
                gb.wait()
                pltpu.sync_copy(rows2, acc_sh.at[dstv2], add=True)

            @pl.when(cb >= nchunk)
            def _():
                ga.wait()
                pltpu.sync_copy(rows, acc_sh.at[dstv], add=True)
        return carry

    lax.fori_loop(0, (nchunk + 2 * NSUB - 1) // (2 * NSUB), pair_step, 0)
    plsc.subcore_barrier()

    def _wb(nrows):
        pltpu.sync_copy(acc_sh.at[pl.ds(r0, nrows)],
                        y_hbm.at[pl.ds(c * N + r0, nrows)])

    @pl.when(s < NSUB - 1)
    def _():
        _wb(R_SMALL)

    @pl.when(s == NSUB - 1)
    def _():
        _wb(R_LAST)


@functools.lru_cache(maxsize=None)
def _sc_agg():
    return pl.kernel(
        _sc_agg_body,
        out_type=jax.ShapeDtypeStruct((NCORE * N, DH), jnp.float32),
        mesh=_mesh(),
        scratch_types=[
            pltpu.VMEM((CH,), jnp.int32),
            pltpu.VMEM((CH,), jnp.int32),
            pltpu.VMEM((CH, DH), jnp.float32),
            pltpu.VMEM((CH,), jnp.int32),
            pltpu.VMEM((CH,), jnp.int32),
            pltpu.VMEM((CH, DH), jnp.float32),
            pltpu.VMEM_SHARED((N, DH), jnp.float32),
            pltpu.SemaphoreType.DMA,
            pltpu.SemaphoreType.DMA,
        ],
    )


def _split_halves(u_ref, u):
    u_ref[0:N, :] = u[:, 0:DH]
    u_ref[N:2 * N, :] = u[:, DH:D]
    u_ref[2 * N:NP, :] = jnp.zeros((NP - 2 * N, DH), jnp.float32)


def _tc_enc_body(x_ref, w_ref, degp_ref, u_ref, dis_ref):
    deg = degp_ref[0:N, 0:1]
    dis = lax.rsqrt(deg)
    dis_ref[...] = dis
    h = jnp.dot(x_ref[...], w_ref[...], preferred_element_type=jnp.float32)
    _split_halves(u_ref, h * dis)


_tc_enc = pl.pallas_call(
    _tc_enc_body,
    out_shape=[jax.ShapeDtypeStruct((NP, DH), jnp.float32),
               jax.ShapeDtypeStruct((N, 1), jnp.float32)],
)


def _tc_mid_body(has_bn, y_ref, dis_ref, b_ref, w_ref, g_ref, be_ref, u_ref):
    dis = dis_ref[...]
    h = jnp.concatenate([y_ref[0:N, :], y_ref[N:2 * N, :]], axis=1)
    h = h * dis + b_ref[...]
    if has_bn:
        m = jnp.mean(h, axis=0, keepdims=True)
        v = jnp.mean((h - m) ** 2, axis=0, keepdims=True)
        h = (h - m) * lax.rsqrt(v + 1e-5) * g_ref[...] + be_ref[...]
        h = jnp.maximum(h, 0.0)
    h2 = jnp.dot(h, w_ref[...], preferred_element_type=jnp.float32)
    _split_halves(u_ref, h2 * dis)


_tc_mid_bn = pl.pallas_call(
    functools.partial(_tc_mid_body, True),
    out_shape=jax.ShapeDtypeStruct((NP, DH), jnp.float32),
)
_tc_mid_plain = pl.pallas_call(
    functools.partial(_tc_mid_body, False),
    out_shape=jax.ShapeDtypeStruct((NP, DH), jnp.float32),
)


def _tc_fin_body(y_ref, dis_ref, b_ref, batch_ref, wc_ref, bc_ref, out_ref):
    dis = dis_ref[...]
    h = jnp.concatenate([y_ref[0:N, :], y_ref[N:2 * N, :]], axis=1)
    h = h * dis + b_ref[...]
    onehot = (batch_ref[...] == lax.broadcasted_iota(jnp.int32, (N, NG), 1)
              ).astype(jnp.float32)
    ssum = lax.dot_general(onehot, h, (((0,), (0,)), ((), ())),
                           preferred_element_type=jnp.float32)
    cnt = jnp.sum(onehot, axis=0, keepdims=True)
    pooled = ssum / jnp.maximum(cnt.T, 1.0)
    out_ref[...] = jnp.dot(pooled, wc_ref[...],
                           preferred_element_type=jnp.float32) + bc_ref[...]


_tc_fin = pl.pallas_call(
    _tc_fin_body,
    out_shape=jax.ShapeDtypeStruct((NG, 64), jnp.float32),
)


def kernel(x, edge_index, batch, edge_attr, W_enc, b_enc, W_convs, b_convs,
           gamma, beta, W_clf, b_clf):
    src = edge_index[0].astype(jnp.int32)
    dst = edge_index[1].astype(jnp.int32)
    npad = EPC - E
    pad_src = jnp.full((npad,), 2 * N, jnp.int32)
    pad_dst = (jnp.arange(npad, dtype=jnp.int32) * 13) % N
    src2 = jnp.concatenate([src, pad_src, src + N, pad_src])
    dst2 = jnp.concatenate([dst, pad_dst])

    sc_agg = _sc_agg()
    ones_u = jnp.concatenate([jnp.ones((2 * N, DH), jnp.float32),
                              jnp.zeros((NP - 2 * N, DH), jnp.float32)])
    degp = sc_agg(ones_u, src2, dst2)
    u, dis = _tc_enc(x, W_enc, degp)

    y = sc_agg(u, src2, dst2)
    u = _tc_mid_plain(y, dis, b_enc.reshape(1, D), W_convs[0],
                      gamma[0].reshape(1, D), beta[0].reshape(1, D))
    y = sc_agg(u, src2, dst2)
    u = _tc_mid_bn(y, dis, b_convs[0].reshape(1, D), W_convs[2],
                   gamma[0].reshape(1, D), beta[0].reshape(1, D))
    y = sc_agg(u, src2, dst2)
    u = _tc_mid_plain(y, dis, b_convs[2].reshape(1, D), W_convs[1],
                      gamma[0].reshape(1, D), beta[0].reshape(1, D))
    y = sc_agg(u, src2, dst2)
    u = _tc_mid_bn(y, dis, b_convs[1].reshape(1, D), W_convs[2],
                   gamma[1].reshape(1, D), beta[1].reshape(1, D))
    y = sc_agg(u, src2, dst2)
    out = _tc_fin(y, dis, b_convs[2].reshape(1, D),
                  batch.astype(jnp.int32).reshape(N, 1), W_clf,
                  b_clf.reshape(1, 64))
    return out


# speedup vs baseline: 2.8521x; 1.1083x over previous
import functools

import jax
import jax.numpy as jnp
from jax import lax
from jax.experimental import pallas as pl
from jax.experimental.pallas import tpu as pltpu
from jax.experimental.pallas import tpu_sc as plsc

N = 10000
E = 160000
D = 256
DH = 128
NG = 16
NCORE = 2
NSUB = 16
CH = 192
EPC = 163840
NP = 2 * N + 16
R_SMALL = 624
R_LAST = N - (NSUB - 1) * R_SMALL

@functools.lru_cache(maxsize=None)
def _mesh():
    return plsc.VectorSubcoreMesh(
        core_axis_name="c", subcore_axis_name="s",
        num_cores=NCORE, num_subcores=NSUB)


def _sc_agg_body(u_hbm, src_hbm, dst_hbm, y_hbm,
                 srcv, dstv, rows, srcv2, dstv2, rows2, acc_sh, sem_a, sem_b):
    c = lax.axis_index("c")
    s = lax.axis_index("s")
    r0 = s * R_SMALL

    def _init(nrows):
        pltpu.sync_copy(u_hbm.at[pl.ds(c * N + r0, nrows)],
                        acc_sh.at[pl.ds(r0, nrows)])

    @pl.when(s < NSUB - 1)
    def _():
        _init(R_SMALL)

    @pl.when(s == NSUB - 1)
    def _():
        _init(R_LAST)

    plsc.subcore_barrier()
    nchunk = (E + CH - 1) // CH

    def _issue(g, sv, dv, rw, sem):
        @pl.when(g * NSUB + s < nchunk)
        def _():
            off = pl.multiple_of((g * NSUB + s) * CH, CH)
            pltpu.sync_copy(src_hbm.at[pl.ds(c * EPC + off, CH)], sv)
            pltpu.sync_copy(dst_hbm.at[pl.ds(off, CH)], dv)
            pltpu.async_copy(u_hbm.at[sv], rw, sem)

    def _drain(g, dv, rw, sem):
        @pl.when(g * NSUB + s < nchunk)
        def _():
            pltpu.make_async_copy(u_hbm.at[dv], rw, sem).wait()
            pltpu.sync_copy(rw, acc_sh.at[dv], add=True)

    ngrp = (nchunk + NSUB - 1) // NSUB
    _issue(0, srcv, dstv, rows, sem_a)

    def pipe_step(j, carry):
        ga, gb = 2 * j, 2 * j + 1
        _issue(gb, srcv2, dstv2, rows2, sem_b)
        _drain(ga, dstv, rows, sem_a)
        _issue(gb + 1, srcv, dstv, rows, sem_a)
        _drain(gb, dstv2, rows2, sem_b)
        return carry

    lax.fori_loop(0, (ngrp + 1) // 2, pipe_step, 0)
    plsc.subcore_barrier()

    def _wb(nrows):
        pltpu.sync_copy(acc_sh.at[pl.ds(r0, nrows)],
                        y_hbm.at[pl.ds(c * N + r0, nrows)])

    @pl.when(s < NSUB - 1)
    def _():
        _wb(R_SMALL)

    @pl.when(s == NSUB - 1)
    def _():
        _wb(R_LAST)


@functools.lru_cache(maxsize=None)
def _sc_agg():
    return pl.kernel(
        _sc_agg_body,
        out_type=jax.ShapeDtypeStruct((NCORE * N, DH), jnp.float32),
        mesh=_mesh(),
        scratch_types=[
            pltpu.VMEM((CH,), jnp.int32),
            pltpu.VMEM((CH,), jnp.int32),
            pltpu.VMEM((CH, DH), jnp.float32),
            pltpu.VMEM((CH,), jnp.int32),
            pltpu.VMEM((CH,), jnp.int32),
            pltpu.VMEM((CH, DH), jnp.float32),
            pltpu.VMEM_SHARED((N, DH), jnp.float32),
            pltpu.SemaphoreType.DMA,
            pltpu.SemaphoreType.DMA,
        ],
    )


def _split_halves(u_ref, u):
    u_ref[0:N, :] = u[:, 0:DH]
    u_ref[N:2 * N, :] = u[:, DH:D]
    u_ref[2 * N:NP, :] = jnp.zeros((NP - 2 * N, DH), jnp.float32)


def _tc_enc_body(x_ref, w_ref, degp_ref, u_ref, dis_ref):
    deg = degp_ref[0:N, 0:1]
    dis = lax.rsqrt(deg)
    dis_ref[...] = dis
    h = jnp.dot(x_ref[...], w_ref[...], preferred_element_type=jnp.float32)
    _split_halves(u_ref, h * dis)


_tc_enc = pl.pallas_call(
    _tc_enc_body,
    out_shape=[jax.ShapeDtypeStruct((NP, DH), jnp.float32),
               jax.ShapeDtypeStruct((N, 1), jnp.float32)],
)


def _tc_mid_body(has_bn, y_ref, dis_ref, b_ref, w_ref, g_ref, be_ref, u_ref):
    dis = dis_ref[...]
    h = jnp.concatenate([y_ref[0:N, :], y_ref[N:2 * N, :]], axis=1)
    h = h * dis + b_ref[...]
    if has_bn:
        m = jnp.mean(h, axis=0, keepdims=True)
        v = jnp.mean((h - m) ** 2, axis=0, keepdims=True)
        h = (h - m) * lax.rsqrt(v + 1e-5) * g_ref[...] + be_ref[...]
        h = jnp.maximum(h, 0.0)
    h2 = jnp.dot(h, w_ref[...], preferred_element_type=jnp.float32)
    _split_halves(u_ref, h2 * dis)


_tc_mid_bn = pl.pallas_call(
    functools.partial(_tc_mid_body, True),
    out_shape=jax.ShapeDtypeStruct((NP, DH), jnp.float32),
)
_tc_mid_plain = pl.pallas_call(
    functools.partial(_tc_mid_body, False),
    out_shape=jax.ShapeDtypeStruct((NP, DH), jnp.float32),
)


def _tc_fin_body(y_ref, dis_ref, b_ref, batch_ref, wc_ref, bc_ref, out_ref):
    dis = dis_ref[...]
    h = jnp.concatenate([y_ref[0:N, :], y_ref[N:2 * N, :]], axis=1)
    h = h * dis + b_ref[...]
    onehot = (batch_ref[...] == lax.broadcasted_iota(jnp.int32, (N, NG), 1)
              ).astype(jnp.float32)
    ssum = lax.dot_general(onehot, h, (((0,), (0,)), ((), ())),
                           preferred_element_type=jnp.float32)
    cnt = jnp.sum(onehot, axis=0, keepdims=True)
    pooled = ssum / jnp.maximum(cnt.T, 1.0)
    out_ref[...] = jnp.dot(pooled, wc_ref[...],
                           preferred_element_type=jnp.float32) + bc_ref[...]


_tc_fin = pl.pallas_call(
    _tc_fin_body,
    out_shape=jax.ShapeDtypeStruct((NG, 64), jnp.float32),
)


def kernel(x, edge_index, batch, edge_attr, W_enc, b_enc, W_convs, b_convs,
           gamma, beta, W_clf, b_clf):
    src = edge_index[0].astype(jnp.int32)
    dst = edge_index[1].astype(jnp.int32)
    npad = EPC - E
    pad_src = jnp.full((npad,), 2 * N, jnp.int32)
    pad_dst = (jnp.arange(npad, dtype=jnp.int32) * 13) % N
    src2 = jnp.concatenate([src, pad_src, src + N, pad_src])
    dst2 = jnp.concatenate([dst, pad_dst])

    sc_agg = _sc_agg()
    ones_u = jnp.concatenate([jnp.ones((2 * N, DH), jnp.float32),
                              jnp.zeros((NP - 2 * N, DH), jnp.float32)])
    degp = sc_agg(ones_u, src2, dst2)
    u, dis = _tc_enc(x, W_enc, degp)

    y = sc_agg(u, src2, dst2)
    u = _tc_mid_plain(y, dis, b_enc.reshape(1, D), W_convs[0],
                      gamma[0].reshape(1, D), beta[0].reshape(1, D))
    y = sc_agg(u, src2, dst2)
    u = _tc_mid_bn(y, dis, b_convs[0].reshape(1, D), W_convs[2],
                   gamma[0].reshape(1, D), beta[0].reshape(1, D))
    y = sc_agg(u, src2, dst2)
    u = _tc_mid_plain(y, dis, b_convs[2].reshape(1, D), W_convs[1],
                      gamma[0].reshape(1, D), beta[0].reshape(1, D))
    y = sc_agg(u, src2, dst2)
    u = _tc_mid_bn(y, dis, b_convs[1].reshape(1, D), W_convs[2],
                   gamma[1].reshape(1, D), beta[1].reshape(1, D))
    y = sc_agg(u, src2, dst2)
    out = _tc_fin(y, dis, b_convs[2].reshape(1, D),
                  batch.astype(jnp.int32).reshape(N, 1), W_clf,
                  b_clf.reshape(1, 64))
    return out


# speedup vs baseline: 3.1969x; 1.1209x over previous
import functools

import jax
import jax.numpy as jnp
from jax import lax
from jax.experimental import pallas as pl
from jax.experimental.pallas import tpu as pltpu
from jax.experimental.pallas import tpu_sc as plsc

N = 10000
E = 160000
D = 256
DH = 128
NG = 16
NCORE = 2
NSUB = 16
CH = 192
EPC = 163840
NP = 2 * N + 16
NA = N + 16
R_SMALL = 624
R_LAST = N - (NSUB - 1) * R_SMALL

@functools.lru_cache(maxsize=None)
def _mesh():
    return plsc.VectorSubcoreMesh(
        core_axis_name="c", subcore_axis_name="s",
        num_cores=NCORE, num_subcores=NSUB)


def _sc_agg_body(u_hbm, src_hbm, dst_hbm, y_hbm,
                 srcv, dstv, rows, srcv2, dstv2, rows2, acc_sh, sem_a, sem_b):
    c = lax.axis_index("c")
    s = lax.axis_index("s")
    r0 = s * R_SMALL

    def _init(nrows):
        pltpu.sync_copy(u_hbm.at[pl.ds(c * N + r0, nrows)],
                        acc_sh.at[pl.ds(r0, nrows)])

    @pl.when(s < NSUB - 1)
    def _():
        _init(R_SMALL)

    @pl.when(s == NSUB - 1)
    def _():
        _init(R_LAST)

    plsc.subcore_barrier()
    nchunk = (E + CH - 1) // CH

    def _issue(g, sv, dv, rw, sem):
        @pl.when(g * NSUB + s < nchunk)
        def _():
            off = pl.multiple_of((g * NSUB + s) * CH, CH)
            pltpu.sync_copy(src_hbm.at[pl.ds(c * EPC + off, CH)], sv)
            pltpu.sync_copy(dst_hbm.at[pl.ds(off, CH)], dv)
            pltpu.async_copy(u_hbm.at[sv], rw, sem)

    def _drain(g, dv, rw, sem):
        @pl.when(g * NSUB + s < nchunk)
        def _():
            pltpu.make_async_copy(u_hbm.at[dv], rw, sem).wait()
            pltpu.sync_copy(rw, acc_sh.at[dv], add=True)

    ngrp = (nchunk + NSUB - 1) // NSUB
    _issue(0, srcv, dstv, rows, sem_a)

    def pipe_step(j, carry):
        ga, gb = 2 * j, 2 * j + 1
        _issue(gb, srcv2, dstv2, rows2, sem_b)
        _drain(ga, dstv, rows, sem_a)
        _issue(gb + 1, srcv, dstv, rows, sem_a)
        _drain(gb, dstv2, rows2, sem_b)
        return carry

    lax.fori_loop(0, (ngrp + 1) // 2, pipe_step, 0)
    plsc.subcore_barrier()

    def _wb(nrows):
        pltpu.sync_copy(acc_sh.at[pl.ds(r0, nrows)],
                        y_hbm.at[pl.ds(c * N + r0, nrows)])

    @pl.when(s < NSUB - 1)
    def _():
        _wb(R_SMALL)

    @pl.when(s == NSUB - 1)
    def _():
        _wb(R_LAST)


@functools.lru_cache(maxsize=None)
def _sc_agg():
    return pl.kernel(
        _sc_agg_body,
        out_type=jax.ShapeDtypeStruct((NCORE * N, DH), jnp.float32),
        mesh=_mesh(),
        scratch_types=[
            pltpu.VMEM((CH,), jnp.int32),
            pltpu.VMEM((CH,), jnp.int32),
            pltpu.VMEM((CH, DH), jnp.float32),
            pltpu.VMEM((CH,), jnp.int32),
            pltpu.VMEM((CH,), jnp.int32),
            pltpu.VMEM((CH, DH), jnp.float32),
            pltpu.VMEM_SHARED((NA, DH), jnp.float32),
            pltpu.SemaphoreType.DMA,
            pltpu.SemaphoreType.DMA,
        ],
    )


def _sc_deg_body(dst_hbm, ones_hbm, zeros_hbm, y_hbm, dstv, ones_v, acc_sh):
    c = lax.axis_index("c")
    s = lax.axis_index("s")
    r0 = s * R_SMALL

    def _init(nrows):
        pltpu.sync_copy(zeros_hbm.at[pl.ds(0, nrows)],
                        acc_sh.at[pl.ds(r0, nrows)])

    @pl.when(s < NSUB - 1)
    def _():
        _init(R_SMALL)

    @pl.when(s == NSUB - 1)
    def _():
        _init(R_LAST)

    pltpu.sync_copy(ones_hbm, ones_v)
    plsc.subcore_barrier()
    nchunk = (E + CH - 1) // CH
    nhalf = (nchunk + 1) // 2

    def chunk_step(j, carry):
        rel = j * NSUB + s
        chunk = c * nhalf + rel

        @pl.when((rel < nhalf) & (chunk < nchunk))
        def _():
            off = pl.multiple_of(chunk * CH, CH)
            pltpu.sync_copy(dst_hbm.at[pl.ds(off, CH)], dstv)
            pltpu.sync_copy(ones_v, acc_sh.at[dstv], add=True)
        return carry

    lax.fori_loop(0, (nhalf + NSUB - 1) // NSUB, chunk_step, 0)
    plsc.subcore_barrier()

    def _wb(nrows):
        pltpu.sync_copy(acc_sh.at[pl.ds(r0, nrows)],
                        y_hbm.at[pl.ds(c * N + r0, nrows)])

    @pl.when(s < NSUB - 1)
    def _():
        _wb(R_SMALL)

    @pl.when(s == NSUB - 1)
    def _():
        _wb(R_LAST)


@functools.lru_cache(maxsize=None)
def _sc_deg():
    return pl.kernel(
        _sc_deg_body,
        out_type=jax.ShapeDtypeStruct((NCORE * N, DH), jnp.float32),
        mesh=_mesh(),
        scratch_types=[
            pltpu.VMEM((CH,), jnp.int32),
            pltpu.VMEM((CH, DH), jnp.float32),
            pltpu.VMEM_SHARED((NA, DH), jnp.float32),
        ],
    )


def _split_halves(u_ref, u):
    u_ref[0:N, :] = u[:, 0:DH]
    u_ref[N:2 * N, :] = u[:, DH:D]
    u_ref[2 * N:NP, :] = jnp.zeros((NP - 2 * N, DH), jnp.float32)


def _tc_enc_body(x_ref, w_ref, degp_ref, u_ref, dis_ref):
    deg = degp_ref[0:N, 0:1] + degp_ref[N:2 * N, 0:1] + 1.0
    dis = lax.rsqrt(deg)
    dis_ref[...] = dis
    h = jnp.dot(x_ref[...], w_ref[...], preferred_element_type=jnp.float32)
    _split_halves(u_ref, h * dis)


_tc_enc = pl.pallas_call(
    _tc_enc_body,
    out_shape=[jax.ShapeDtypeStruct((NP, DH), jnp.float32),
               jax.ShapeDtypeStruct((N, 1), jnp.float32)],
)


def _tc_mid_body(has_bn, y_ref, dis_ref, b_ref, w_ref, g_ref, be_ref, u_ref):
    dis = dis_ref[...]
    h = jnp.concatenate([y_ref[0:N, :], y_ref[N:2 * N, :]], axis=1)
    h = h * dis + b_ref[...]
    if has_bn:
        m = jnp.mean(h, axis=0, keepdims=True)
        v = jnp.mean((h - m) ** 2, axis=0, keepdims=True)
        h = (h - m) * lax.rsqrt(v + 1e-5) * g_ref[...] + be_ref[...]
        h = jnp.maximum(h, 0.0)
    h2 = jnp.dot(h, w_ref[...], preferred_element_type=jnp.float32)
    _split_halves(u_ref, h2 * dis)


_tc_mid_bn = pl.pallas_call(
    functools.partial(_tc_mid_body, True),
    out_shape=jax.ShapeDtypeStruct((NP, DH), jnp.float32),
)
_tc_mid_plain = pl.pallas_call(
    functools.partial(_tc_mid_body, False),
    out_shape=jax.ShapeDtypeStruct((NP, DH), jnp.float32),
)


def _tc_fin_body(y_ref, dis_ref, b_ref, batch_ref, wc_ref, bc_ref, out_ref):
    dis = dis_ref[...]
    h = jnp.concatenate([y_ref[0:N, :], y_ref[N:2 * N, :]], axis=1)
    h = h * dis + b_ref[...]
    onehot = (batch_ref[...] == lax.broadcasted_iota(jnp.int32, (N, NG), 1)
              ).astype(jnp.float32)
    ssum = lax.dot_general(onehot, h, (((0,), (0,)), ((), ())),
                           preferred_element_type=jnp.float32)
    cnt = jnp.sum(onehot, axis=0, keepdims=True)
    pooled = ssum / jnp.maximum(cnt.T, 1.0)
    out_ref[...] = jnp.dot(pooled, wc_ref[...],
                           preferred_element_type=jnp.float32) + bc_ref[...]


_tc_fin = pl.pallas_call(
    _tc_fin_body,
    out_shape=jax.ShapeDtypeStruct((NG, 64), jnp.float32),
)


def kernel(x, edge_index, batch, edge_attr, W_enc, b_enc, W_convs, b_convs,
           gamma, beta, W_clf, b_clf):
    src = edge_index[0].astype(jnp.int32)
    dst = edge_index[1].astype(jnp.int32)
    npad = EPC - E
    pad_src = jnp.full((npad,), 2 * N, jnp.int32)
    pad_dst = jnp.full((npad,), N, jnp.int32)
    src2 = jnp.concatenate([src, pad_src, src + N, pad_src])
    dst2 = jnp.concatenate([dst, pad_dst])

    sc_agg, sc_deg = _sc_agg(), _sc_deg()
    degp = sc_deg(dst2, jnp.ones((CH, DH), jnp.float32),
                  jnp.zeros((R_LAST, DH), jnp.float32))
    u, dis = _tc_enc(x, W_enc, degp)

    y = sc_agg(u, src2, dst2)
    u = _tc_mid_plain(y, dis, b_enc.reshape(1, D), W_convs[0],
                      gamma[0].reshape(1, D), beta[0].reshape(1, D))
    y = sc_agg(u, src2, dst2)
    u = _tc_mid_bn(y, dis, b_convs[0].reshape(1, D), W_convs[2],
                   gamma[0].reshape(1, D), beta[0].reshape(1, D))
    y = sc_agg(u, src2, dst2)
    u = _tc_mid_plain(y, dis, b_convs[2].reshape(1, D), W_convs[1],
                      gamma[0].reshape(1, D), beta[0].reshape(1, D))
    y = sc_agg(u, src2, dst2)
    u = _tc_mid_bn(y, dis, b_convs[1].reshape(1, D), W_convs[2],
                   gamma[1].reshape(1, D), beta[1].reshape(1, D))
    y = sc_agg(u, src2, dst2)
    out = _tc_fin(y, dis, b_convs[2].reshape(1, D),
                  batch.astype(jnp.int32).reshape(N, 1), W_clf,
                  b_clf.reshape(1, 64))
    return out
